# Initial kernel scaffold; baseline (speedup 1.0000x reference)
import jax
import jax.numpy as jnp
from jax.experimental import pallas as pl


def kernel(grid_feat, grid_coord):
    raise NotImplementedError("write your pallas kernel here")


# speedup vs baseline: 1.1260x; 1.1260x over previous
import functools

import jax
import jax.numpy as jnp
from jax import lax
from jax.experimental import pallas as pl
from jax.experimental.pallas import tpu as pltpu
from jax.experimental.pallas import tpu_sc as plsc

B, C, H, W = 4, 96, 224, 224
N, S = 20000, 4
NS = N * S
HW = H * W
NPLANES = B * C
SCALE_X = 223.0
SCALE_Y = 223.0

NC, NSUB = 2, 16
NW = NC * NSUB
TILES_PER_BATCH = NW // B
CPT = C // TILES_PER_BATCH
CH = 3200
NCH = NS // CH


def _coef_body(cx_ref, cy_ref, idx_ref, wx_ref, wy_ref):
    cx = cx_ref[...]
    cy = cy_ref[...]
    gx = 2.0 * cx * SCALE_X / (W - 1) - 1.0
    gy = 2.0 * cy * SCALE_Y / (H - 1) - 1.0
    ix = (gx + 1.0) * 0.5 * (W - 1)
    iy = (gy + 1.0) * 0.5 * (H - 1)
    ix0 = jnp.floor(ix)
    iy0 = jnp.floor(iy)
    wx_ref[...] = ix - ix0
    wy_ref[...] = iy - iy0
    ix0i = jnp.clip(ix0.astype(jnp.int32), 0, W - 2)
    iy0i = jnp.clip(iy0.astype(jnp.int32), 0, H - 2)
    idx_ref[...] = iy0i * W + ix0i


_coef = pl.pallas_call(
    _coef_body,
    out_shape=(
        jax.ShapeDtypeStruct((B, NS), jnp.int32),
        jax.ShapeDtypeStruct((B, NS), jnp.float32),
        jax.ShapeDtypeStruct((B, NS), jnp.float32),
    ),
)


@functools.partial(
    pl.kernel,
    mesh=plsc.VectorSubcoreMesh(core_axis_name="c", subcore_axis_name="s"),
    out_type=jax.ShapeDtypeStruct((NPLANES * NS,), jnp.float32),
    compiler_params=pltpu.CompilerParams(needs_layout_passes=False),
    scratch_types=[
        pltpu.VMEM((HW,), jnp.float32),
        pltpu.VMEM((HW,), jnp.float32),
        pltpu.VMEM((CH,), jnp.int32),
        pltpu.VMEM((CH,), jnp.float32),
        pltpu.VMEM((CH,), jnp.float32),
        pltpu.VMEM((CH,), jnp.float32),
        pltpu.VMEM((CH,), jnp.float32),
    ],
)
def _sc_sample(feat_hbm, idx_hbm, wx_hbm, wy_hbm, out_hbm,
               plane0_v, plane1_v, idx_v, wx_v, wy_v, out0_v, out1_v):
    wid = lax.axis_index("s") * NC + lax.axis_index("c")
    batch = wid // TILES_PER_BATCH
    p_base = batch * C + (wid % TILES_PER_BATCH) * CPT

    def plane_pair(pp, carry):
        p0 = p_base + 2 * pp
        pltpu.sync_copy(feat_hbm.at[pl.ds(p0 * HW, HW)], plane0_v)
        pltpu.sync_copy(feat_hbm.at[pl.ds((p0 + 1) * HW, HW)], plane1_v)

        def chunk(ck, carry2):
            off = batch * NS + ck * CH
            pltpu.sync_copy(idx_hbm.at[pl.ds(off, CH)], idx_v)
            pltpu.sync_copy(wx_hbm.at[pl.ds(off, CH)], wx_v)
            pltpu.sync_copy(wy_hbm.at[pl.ds(off, CH)], wy_v)

            def body(i, carry3):
                sl = pl.ds(i * 16, 16)
                i00 = idx_v[sl]
                wx1 = wx_v[sl]
                wy1 = wy_v[sl]
                wx0 = 1.0 - wx1
                wy0 = 1.0 - wy1
                w00 = wx0 * wy0
                w10 = wx1 * wy0
                w01 = wx0 * wy1
                w11 = wx1 * wy1
                i10 = i00 + 1
                i01 = i00 + W
                i11 = i00 + (W + 1)
                for plane_v, ov in ((plane0_v, out0_v), (plane1_v, out1_v)):
                    g00 = plsc.load_gather(plane_v, [i00])
                    g10 = plsc.load_gather(plane_v, [i10])
                    g01 = plsc.load_gather(plane_v, [i01])
                    g11 = plsc.load_gather(plane_v, [i11])
                    ov[sl] = g00 * w00 + g10 * w10 + g01 * w01 + g11 * w11
                return carry3

            lax.fori_loop(0, CH // 16, body, 0)
            ooff = ck * CH
            pltpu.sync_copy(out0_v, out_hbm.at[pl.ds(p0 * NS + ooff, CH)])
            pltpu.sync_copy(out1_v, out_hbm.at[pl.ds((p0 + 1) * NS + ooff, CH)])
            return carry2

        lax.fori_loop(0, NCH, chunk, 0)
        return carry

    lax.fori_loop(0, CPT // 2, plane_pair, 0)


def kernel(grid_feat, grid_coord):
    cx = grid_coord[:, :, 1, :].reshape(B, NS)
    cy = grid_coord[:, :, 0, :].reshape(B, NS)
    idx, wx1, wy1 = _coef(cx, cy)
    feat_flat = grid_feat.reshape(NPLANES * HW)
    out_flat = _sc_sample(feat_flat, idx.reshape(B * NS), wx1.reshape(B * NS),
                          wy1.reshape(B * NS))
    return out_flat.reshape(B, C, N, S)


# speedup vs baseline: 4.8491x; 4.3066x over previous
import functools

import jax
import jax.numpy as jnp
from jax import lax
from jax.experimental import pallas as pl
from jax.experimental.pallas import tpu as pltpu
from jax.experimental.pallas import tpu_sc as plsc

B, C, H, W = 4, 96, 224, 224
N, S = 20000, 4
NS = N * S
NPLANES = B * C
SCALE_X = 223.0
SCALE_Y = 223.0

NC, NSUB = 2, 16
NW = NC * NSUB
TILES_PER_BATCH = NW // B
CPT = C // TILES_PER_BATCH

CHN = 512
CH = CHN * S
NCH = N // CHN
CHNT = N - NCH * CHN
CHT = CHNT * S


def _coef_body(cx_ref, cy_ref, pk_ref, wx_ref, wy_ref):
    cx = cx_ref[...]
    cy = cy_ref[...]
    gx = 2.0 * cx * SCALE_X / (W - 1) - 1.0
    gy = 2.0 * cy * SCALE_Y / (H - 1) - 1.0
    ix = (gx + 1.0) * 0.5 * (W - 1)
    iy = (gy + 1.0) * 0.5 * (H - 1)
    ix0 = jnp.floor(ix)
    iy0 = jnp.floor(iy)
    wx_ref[...] = ix - ix0
    wy_ref[...] = iy - iy0
    ix0i = jnp.clip(ix0.astype(jnp.int32), 0, W - 2)
    iy0i = jnp.clip(iy0.astype(jnp.int32), 0, H - 2)
    pk_ref[...] = iy0i * 256 + ix0i


_coef = pl.pallas_call(
    _coef_body,
    out_shape=(
        jax.ShapeDtypeStruct((B * NS,), jnp.int32),
        jax.ShapeDtypeStruct((B * NS,), jnp.float32),
        jax.ShapeDtypeStruct((B * NS,), jnp.float32),
    ),
)


@functools.partial(
    pl.kernel,
    mesh=plsc.VectorSubcoreMesh(core_axis_name="c", subcore_axis_name="s"),
    out_type=jax.ShapeDtypeStruct((B, C, S, N), jnp.float32),
    compiler_params=pltpu.CompilerParams(needs_layout_passes=False),
    scratch_types=[
        pltpu.VMEM((H, W), jnp.float32),
        pltpu.VMEM((H, W), jnp.float32),
        pltpu.VMEM((CH,), jnp.int32),
        pltpu.VMEM((CH,), jnp.float32),
        pltpu.VMEM((CH,), jnp.float32),
        pltpu.VMEM((S, CHN), jnp.float32),
        pltpu.VMEM((S, CHN), jnp.float32),
        pltpu.VMEM((S, CHNT), jnp.float32),
        pltpu.VMEM((S, CHNT), jnp.float32),
    ],
)
def _sc_sample(feat_hbm, pk_hbm, wx_hbm, wy_hbm, out_hbm,
               plane0_v, plane1_v, pk_v, wx_v, wy_v,
               om0_v, om1_v, ot0_v, ot1_v):
    wid = lax.axis_index("s") * NC + lax.axis_index("c")
    batch = wid // TILES_PER_BATCH
    c_base = (wid % TILES_PER_BATCH) * CPT
    iota = lax.iota(jnp.int32, 16)
    s_vec = jnp.bitwise_and(iota, 3)
    n_off = lax.shift_right_logical(iota, 2)

    def make_body(outs):
        def body(i, carry):
            sl = pl.ds(i * 16, 16)
            pk = pk_v[sl]
            wx1 = wx_v[sl]
            wy1 = wy_v[sl]
            iy0 = lax.shift_right_logical(pk, 8)
            ix0 = jnp.bitwise_and(pk, 255)
            iy1 = iy0 + 1
            ix1 = ix0 + 1
            wx0 = 1.0 - wx1
            wy0 = 1.0 - wy1
            w00 = wx0 * wy0
            w10 = wx1 * wy0
            w01 = wx0 * wy1
            w11 = wx1 * wy1
            n_vec = n_off + i * 4
            for plane_v, ov in outs:
                g00 = plsc.load_gather(plane_v, [iy0, ix0])
                g10 = plsc.load_gather(plane_v, [iy0, ix1])
                g01 = plsc.load_gather(plane_v, [iy1, ix0])
                g11 = plsc.load_gather(plane_v, [iy1, ix1])
                acc = g00 * w00 + g10 * w10 + g01 * w01 + g11 * w11
                plsc.store_scatter(ov, [s_vec, n_vec], acc)
            return carry
        return body

    main_body = make_body(((plane0_v, om0_v), (plane1_v, om1_v)))
    tail_body = make_body(((plane0_v, ot0_v), (plane1_v, ot1_v)))

    def plane_pair(pp, carry):
        c0 = c_base + 2 * pp
        pltpu.sync_copy(feat_hbm.at[batch, c0], plane0_v)
        pltpu.sync_copy(feat_hbm.at[batch, c0 + 1], plane1_v)

        def chunk(ck, carry2):
            off = batch * NS + ck * CH
            pltpu.sync_copy(pk_hbm.at[pl.ds(off, CH)], pk_v)
            pltpu.sync_copy(wx_hbm.at[pl.ds(off, CH)], wx_v)
            pltpu.sync_copy(wy_hbm.at[pl.ds(off, CH)], wy_v)
            lax.fori_loop(0, CH // 16, main_body, 0)
            n0 = ck * CHN
            pltpu.sync_copy(om0_v, out_hbm.at[batch, c0, :, pl.ds(n0, CHN)])
            pltpu.sync_copy(om1_v, out_hbm.at[batch, c0 + 1, :, pl.ds(n0, CHN)])
            return carry2

        lax.fori_loop(0, NCH, chunk, 0)

        offt = batch * NS + NCH * CH
        pltpu.sync_copy(pk_hbm.at[pl.ds(offt, CHT)], pk_v.at[pl.ds(0, CHT)])
        pltpu.sync_copy(wx_hbm.at[pl.ds(offt, CHT)], wx_v.at[pl.ds(0, CHT)])
        pltpu.sync_copy(wy_hbm.at[pl.ds(offt, CHT)], wy_v.at[pl.ds(0, CHT)])
        lax.fori_loop(0, CHT // 16, tail_body, 0)
        nt = NCH * CHN
        pltpu.sync_copy(ot0_v, out_hbm.at[batch, c0, :, pl.ds(nt, CHNT)])
        pltpu.sync_copy(ot1_v, out_hbm.at[batch, c0 + 1, :, pl.ds(nt, CHNT)])
        return carry

    lax.fori_loop(0, CPT // 2, plane_pair, 0)


def kernel(grid_feat, grid_coord):
    cx = grid_coord[:, :, 1, :].reshape(B * NS)
    cy = grid_coord[:, :, 0, :].reshape(B * NS)
    pk, wx1, wy1 = _coef(cx, cy)
    out_scn = _sc_sample(grid_feat, pk, wx1, wy1)
    return jnp.transpose(out_scn, (0, 1, 3, 2))


# speedup vs baseline: 5.8446x; 1.2053x over previous
import functools

import jax
import jax.numpy as jnp
from jax import lax
from jax.experimental import pallas as pl
from jax.experimental.pallas import tpu as pltpu
from jax.experimental.pallas import tpu_sc as plsc

B, C, H, W = 4, 96, 224, 224
N, S = 20000, 4
NS = N * S
NPLANES = B * C
SCALE_X = 223.0
SCALE_Y = 223.0

NC, NSUB = 2, 16
NW = NC * NSUB
TILES_PER_BATCH = NW // B
CPT = C // TILES_PER_BATCH

CHN = 512
CH = CHN * S
NCH = N // CHN
CHNT = N - NCH * CHN
CHT = CHNT * S


def _coef_body(cx_ref, cy_ref, pk_ref, wx_ref, wy_ref):
    cx = cx_ref[...]
    cy = cy_ref[...]
    gx = 2.0 * cx * SCALE_X / (W - 1) - 1.0
    gy = 2.0 * cy * SCALE_Y / (H - 1) - 1.0
    ix = (gx + 1.0) * 0.5 * (W - 1)
    iy = (gy + 1.0) * 0.5 * (H - 1)
    ix0 = jnp.floor(ix)
    iy0 = jnp.floor(iy)
    wx_ref[...] = ix - ix0
    wy_ref[...] = iy - iy0
    ix0i = jnp.clip(ix0.astype(jnp.int32), 0, W - 2)
    iy0i = jnp.clip(iy0.astype(jnp.int32), 0, H - 2)
    pk_ref[...] = iy0i * 256 + ix0i


_coef = pl.pallas_call(
    _coef_body,
    out_shape=(
        jax.ShapeDtypeStruct((B * NS,), jnp.int32),
        jax.ShapeDtypeStruct((B * NS,), jnp.float32),
        jax.ShapeDtypeStruct((B * NS,), jnp.float32),
    ),
)


@functools.partial(
    pl.kernel,
    mesh=plsc.VectorSubcoreMesh(core_axis_name="c", subcore_axis_name="s"),
    out_type=jax.ShapeDtypeStruct((B, C, S, N), jnp.float32),
    compiler_params=pltpu.CompilerParams(needs_layout_passes=False),
    scratch_types=[
        pltpu.VMEM((H, W), jnp.float32),
        pltpu.VMEM((H, W), jnp.float32),
        pltpu.VMEM((CH,), jnp.int32),
        pltpu.VMEM((CH,), jnp.float32),
        pltpu.VMEM((CH,), jnp.float32),
        pltpu.VMEM((S, CHN), jnp.float32),
        pltpu.VMEM((S, CHN), jnp.float32),
        pltpu.VMEM((S, CHNT), jnp.float32),
        pltpu.VMEM((S, CHNT), jnp.float32),
    ],
)
def _sc_sample(feat_hbm, pk_hbm, wx_hbm, wy_hbm, out_hbm,
               plane0_v, plane1_v, pk_v, wx_v, wy_v,
               om0_v, om1_v, ot0_v, ot1_v):
    wid = lax.axis_index("s") * NC + lax.axis_index("c")
    batch = wid // TILES_PER_BATCH
    c_base = (wid % TILES_PER_BATCH) * CPT
    iota = lax.iota(jnp.int32, 16)
    s_vec = jnp.bitwise_and(iota, 3)
    n_off = lax.shift_right_logical(iota, 2)

    def make_body(outs):
        def body(i):
            sl = pl.ds(i * 16, 16)
            pk = pk_v[sl]
            wx1 = wx_v[sl]
            wy1 = wy_v[sl]
            iy0 = lax.shift_right_logical(pk, 8)
            ix0 = jnp.bitwise_and(pk, 255)
            iy1 = iy0 + 1
            ix1 = ix0 + 1
            wx0 = 1.0 - wx1
            wy0 = 1.0 - wy1
            w00 = wx0 * wy0
            w10 = wx1 * wy0
            w01 = wx0 * wy1
            w11 = wx1 * wy1
            n_vec = n_off + i * 4
            for plane_v, ov in outs:
                g00 = plsc.load_gather(plane_v, [iy0, ix0])
                g10 = plsc.load_gather(plane_v, [iy0, ix1])
                g01 = plsc.load_gather(plane_v, [iy1, ix0])
                g11 = plsc.load_gather(plane_v, [iy1, ix1])
                acc = g00 * w00 + g10 * w10 + g01 * w01 + g11 * w11
                plsc.store_scatter(ov, [s_vec, n_vec], acc)
        return body

    main_body = make_body(((plane0_v, om0_v), (plane1_v, om1_v)))
    tail_body = make_body(((plane0_v, ot0_v), (plane1_v, ot1_v)))

    def plane_pair(pp, carry):
        c0 = c_base + 2 * pp
        pltpu.sync_copy(feat_hbm.at[batch, c0], plane0_v)
        pltpu.sync_copy(feat_hbm.at[batch, c0 + 1], plane1_v)

        def chunk(ck, carry2):
            off = batch * NS + ck * CH
            pltpu.sync_copy(pk_hbm.at[pl.ds(off, CH)], pk_v)
            pltpu.sync_copy(wx_hbm.at[pl.ds(off, CH)], wx_v)
            pltpu.sync_copy(wy_hbm.at[pl.ds(off, CH)], wy_v)
            plsc.parallel_loop(0, CH // 16, unroll=4)(main_body)
            n0 = ck * CHN
            pltpu.sync_copy(om0_v, out_hbm.at[batch, c0, :, pl.ds(n0, CHN)])
            pltpu.sync_copy(om1_v, out_hbm.at[batch, c0 + 1, :, pl.ds(n0, CHN)])
            return carry2

        lax.fori_loop(0, NCH, chunk, 0)

        offt = batch * NS + NCH * CH
        pltpu.sync_copy(pk_hbm.at[pl.ds(offt, CHT)], pk_v.at[pl.ds(0, CHT)])
        pltpu.sync_copy(wx_hbm.at[pl.ds(offt, CHT)], wx_v.at[pl.ds(0, CHT)])
        pltpu.sync_copy(wy_hbm.at[pl.ds(offt, CHT)], wy_v.at[pl.ds(0, CHT)])
        plsc.parallel_loop(0, CHT // 16, unroll=4)(tail_body)
        nt = NCH * CHN
        pltpu.sync_copy(ot0_v, out_hbm.at[batch, c0, :, pl.ds(nt, CHNT)])
        pltpu.sync_copy(ot1_v, out_hbm.at[batch, c0 + 1, :, pl.ds(nt, CHNT)])
        return carry

    lax.fori_loop(0, CPT // 2, plane_pair, 0)


def kernel(grid_feat, grid_coord):
    cx = grid_coord[:, :, 1, :].reshape(B * NS)
    cy = grid_coord[:, :, 0, :].reshape(B * NS)
    pk, wx1, wy1 = _coef(cx, cy)
    out_scn = _sc_sample(grid_feat, pk, wx1, wy1)
    return jnp.transpose(out_scn, (0, 1, 3, 2))


# speedup vs baseline: 9.7963x; 1.6761x over previous
import functools

import jax
import jax.numpy as jnp
from jax import lax
from jax.experimental import pallas as pl
from jax.experimental.pallas import tpu as pltpu
from jax.experimental.pallas import tpu_sc as plsc

B, C, H, W = 4, 96, 224, 224
N, S = 20000, 4
NS = N * S
NPLANES = B * C
SCALE_X = 223.0
SCALE_Y = 223.0

NC, NSUB = 2, 16
NW = NC * NSUB
TILES_PER_BATCH = NW // B
CPT = C // TILES_PER_BATCH

CHN = 256
CH = CHN * S
NCH = N // CHN
CHNT = N - NCH * CHN
CHT = CHNT * S


def _coef_body(cx_ref, cy_ref, pk_ref, wx_ref, wy_ref):
    cx = cx_ref[...]
    cy = cy_ref[...]
    gx = 2.0 * cx * SCALE_X / (W - 1) - 1.0
    gy = 2.0 * cy * SCALE_Y / (H - 1) - 1.0
    ix = (gx + 1.0) * 0.5 * (W - 1)
    iy = (gy + 1.0) * 0.5 * (H - 1)
    ix0 = jnp.floor(ix)
    iy0 = jnp.floor(iy)
    wx_ref[...] = ix - ix0
    wy_ref[...] = iy - iy0
    ix0i = jnp.clip(ix0.astype(jnp.int32), 0, W - 2)
    iy0i = jnp.clip(iy0.astype(jnp.int32), 0, H - 2)
    pk_ref[...] = iy0i * 256 + ix0i


_coef = pl.pallas_call(
    _coef_body,
    out_shape=(
        jax.ShapeDtypeStruct((B * NS,), jnp.int32),
        jax.ShapeDtypeStruct((B * NS,), jnp.float32),
        jax.ShapeDtypeStruct((B * NS,), jnp.float32),
    ),
)


@functools.partial(
    pl.kernel,
    mesh=plsc.VectorSubcoreMesh(core_axis_name="c", subcore_axis_name="s"),
    out_type=jax.ShapeDtypeStruct((B, C, S, N), jnp.float32),
    compiler_params=pltpu.CompilerParams(needs_layout_passes=False),
    scratch_types=[
        pltpu.VMEM((H, W), jnp.float32),
        pltpu.VMEM((H, W), jnp.float32),
        pltpu.VMEM((CH,), jnp.int32),
        pltpu.VMEM((CH,), jnp.int32),
        pltpu.VMEM((CH,), jnp.float32),
        pltpu.VMEM((CH,), jnp.float32),
        pltpu.VMEM((CH,), jnp.float32),
        pltpu.VMEM((CH,), jnp.float32),
        pltpu.VMEM((S, CHN), jnp.float32),
        pltpu.VMEM((S, CHN), jnp.float32),
        pltpu.VMEM((S, CHN), jnp.float32),
        pltpu.VMEM((S, CHN), jnp.float32),
        pltpu.VMEM((S, CHNT), jnp.float32),
        pltpu.VMEM((S, CHNT), jnp.float32),
        pltpu.SemaphoreType.DMA,
        pltpu.SemaphoreType.DMA,
        pltpu.SemaphoreType.DMA,
        pltpu.SemaphoreType.DMA,
    ],
)
def _sc_sample(feat_hbm, pk_hbm, wx_hbm, wy_hbm, out_hbm,
               plane0_v, plane1_v, pk0_v, pk1_v, wx0_v, wx1_v, wy0_v, wy1_v,
               om00_v, om01_v, om10_v, om11_v, ot0_v, ot1_v,
               sem_in0, sem_in1, sem_out0, sem_out1):
    wid = lax.axis_index("s") * NC + lax.axis_index("c")
    batch = wid // TILES_PER_BATCH
    c_base = (wid % TILES_PER_BATCH) * CPT
    iota = lax.iota(jnp.int32, 16)
    s_vec = jnp.bitwise_and(iota, 3)
    n_off = lax.shift_right_logical(iota, 2)

    pk_v = (pk0_v, pk1_v)
    wx_v = (wx0_v, wx1_v)
    wy_v = (wy0_v, wy1_v)
    om0_v = (om00_v, om01_v)
    om1_v = (om10_v, om11_v)
    sem_in = (sem_in0, sem_in1)
    sem_out = (sem_out0, sem_out1)

    def coef_start(ck, p):
        off = batch * NS + ck * CH
        pltpu.async_copy(pk_hbm.at[pl.ds(off, CH)], pk_v[p], sem_in[p])
        pltpu.async_copy(wx_hbm.at[pl.ds(off, CH)], wx_v[p], sem_in[p])
        pltpu.async_copy(wy_hbm.at[pl.ds(off, CH)], wy_v[p], sem_in[p])

    def coef_wait(p):
        pltpu.make_async_copy(pk_hbm.at[pl.ds(0, CH)], pk_v[p], sem_in[p]).wait()
        pltpu.make_async_copy(wx_hbm.at[pl.ds(0, CH)], wx_v[p], sem_in[p]).wait()
        pltpu.make_async_copy(wy_hbm.at[pl.ds(0, CH)], wy_v[p], sem_in[p]).wait()

    def out_wait(c0, p):
        pltpu.make_async_copy(
            om0_v[p], out_hbm.at[batch, c0, :, pl.ds(0, CHN)], sem_out[p]).wait()
        pltpu.make_async_copy(
            om1_v[p], out_hbm.at[batch, c0 + 1, :, pl.ds(0, CHN)], sem_out[p]).wait()

    def make_body(pk_r, wx_r, wy_r, outs):
        def body(i):
            sl = pl.ds(i * 16, 16)
            pk = pk_r[sl]
            wx1 = wx_r[sl]
            wy1 = wy_r[sl]
            iy0 = lax.shift_right_logical(pk, 8)
            ix0 = jnp.bitwise_and(pk, 255)
            iy1 = iy0 + 1
            ix1 = ix0 + 1
            wx0 = 1.0 - wx1
            wy0 = 1.0 - wy1
            w00 = wx0 * wy0
            w10 = wx1 * wy0
            w01 = wx0 * wy1
            w11 = wx1 * wy1
            n_vec = n_off + i * 4
            for plane_v, ov in outs:
                g00 = plsc.load_gather(plane_v, [iy0, ix0])
                g10 = plsc.load_gather(plane_v, [iy0, ix1])
                g01 = plsc.load_gather(plane_v, [iy1, ix0])
                g11 = plsc.load_gather(plane_v, [iy1, ix1])
                acc = g00 * w00 + g10 * w10 + g01 * w01 + g11 * w11
                plsc.store_scatter(ov, [s_vec, n_vec], acc)
        return body

    def plane_pair(pp, carry):
        c0 = c_base + 2 * pp
        pltpu.sync_copy(feat_hbm.at[batch, c0], plane0_v)
        pltpu.sync_copy(feat_hbm.at[batch, c0 + 1], plane1_v)
        coef_start(0, 0)
        coef_start(1, 1)

        def chunk2(g, carry2):
            for p in (0, 1):
                ck = 2 * g + p
                coef_wait(p)

                @pl.when(g > 0)
                def _():
                    out_wait(c0, p)

                body = make_body(
                    pk_v[p], wx_v[p], wy_v[p],
                    ((plane0_v, om0_v[p]), (plane1_v, om1_v[p])))
                plsc.parallel_loop(0, CH // 16, unroll=4)(body)

                @pl.when(ck + 2 < NCH)
                def _():
                    coef_start(ck + 2, p)

                n0 = ck * CHN
                pltpu.async_copy(
                    om0_v[p], out_hbm.at[batch, c0, :, pl.ds(n0, CHN)],
                    sem_out[p])
                pltpu.async_copy(
                    om1_v[p], out_hbm.at[batch, c0 + 1, :, pl.ds(n0, CHN)],
                    sem_out[p])
            return carry2

        lax.fori_loop(0, NCH // 2, chunk2, 0)

        offt = batch * NS + NCH * CH
        pltpu.sync_copy(pk_hbm.at[pl.ds(offt, CHT)], pk0_v.at[pl.ds(0, CHT)])
        pltpu.sync_copy(wx_hbm.at[pl.ds(offt, CHT)], wx0_v.at[pl.ds(0, CHT)])
        pltpu.sync_copy(wy_hbm.at[pl.ds(offt, CHT)], wy0_v.at[pl.ds(0, CHT)])
        tail_body = make_body(
            pk0_v, wx0_v, wy0_v,
            ((plane0_v, ot0_v), (plane1_v, ot1_v)))
        plsc.parallel_loop(0, CHT // 16, unroll=4)(tail_body)
        nt = NCH * CHN
        pltpu.sync_copy(ot0_v, out_hbm.at[batch, c0, :, pl.ds(nt, CHNT)])
        pltpu.sync_copy(ot1_v, out_hbm.at[batch, c0 + 1, :, pl.ds(nt, CHNT)])
        out_wait(c0, 0)
        out_wait(c0, 1)
        return carry

    lax.fori_loop(0, CPT // 2, plane_pair, 0)


def kernel(grid_feat, grid_coord):
    cx = grid_coord[:, :, 1, :].reshape(B * NS)
    cy = grid_coord[:, :, 0, :].reshape(B * NS)
    pk, wx1, wy1 = _coef(cx, cy)
    out_scn = _sc_sample(grid_feat, pk, wx1, wy1)
    return jnp.transpose(out_scn, (0, 1, 3, 2))
